# baseline (speedup 1.0000x reference)
import functools

import jax
import jax.numpy as jnp
from jax import lax
from jax.experimental import pallas as pl
from jax.experimental.pallas import tpu as pltpu
from jax.experimental.pallas import tpu_sc as plsc

_TB = 1536
_SUB = 3
_TBD = 2304

_INV_SQRT2 = 0.7071067811865476


def _gelu_exact(v):
    return 0.5 * v * (1.0 + jax.lax.erf(v * _INV_SQRT2))


def _encode_body(nsteps, inv_count,
                 x_ref, w1t_ref, b1_ref, w2t_ref, b2_ref, cbt_ref, cbsq_ref,
                 idx_ref, loss_ref):
    i = pl.program_id(0)
    cs = _TB // _SUB
    parts = []
    for c in range(_SUB):
        rows = pl.ds(c * cs, cs)
        h = _gelu_exact(jnp.dot(x_ref[rows, :], w1t_ref[...],
                                preferred_element_type=jnp.float32) + b1_ref[...])
        z = jnp.dot(h, w2t_ref[...],
                    preferred_element_type=jnp.float32) + b2_ref[...]

        zsq = jnp.sum(z * z, axis=1, keepdims=True)
        zc2 = jnp.dot(2.0 * z, cbt_ref[...], preferred_element_type=jnp.float32)
        dist = (zsq + cbsq_ref[...]) - zc2

        m = jnp.min(dist, axis=1, keepdims=True)
        col = jax.lax.broadcasted_iota(jnp.int32, dist.shape, 1)
        idx = jnp.min(jnp.where(dist <= m, col, dist.shape[1]), axis=1)
        idx_ref[0, 0, rows] = idx
        parts.append(jnp.sum(m))

    part = sum(parts).reshape(1, 1)

    @pl.when(i == 0)
    def _():
        loss_ref[...] = jnp.zeros_like(loss_ref)

    loss_ref[...] += part

    @pl.when(i == nsteps - 1)
    def _():
        loss_ref[...] = loss_ref[...] * (1.25 * inv_count)


def _encode(x2, W1, b1, W2, b2, codebook):
    T, D = x2.shape
    cb_size, cb_dim = codebook.shape
    nsteps = T // _TB
    cbsq = jnp.sum(codebook * codebook, axis=1).reshape(1, cb_size)
    full = lambda i: (0, 0)
    grid_spec = pl.GridSpec(
        grid=(nsteps,),
        in_specs=[
            pl.BlockSpec((_TB, D), lambda i: (i, 0)),
            pl.BlockSpec((D, W1.shape[0]), full),
            pl.BlockSpec((1, W1.shape[0]), full),
            pl.BlockSpec((W1.shape[0], cb_dim), full),
            pl.BlockSpec((1, cb_dim), full),
            pl.BlockSpec((cb_dim, cb_size), full),
            pl.BlockSpec((1, cb_size), full),
        ],
        out_specs=[
            pl.BlockSpec((1, 1, _TB), lambda i: (i, 0, 0)),
            pl.BlockSpec((1, 1), full),
        ],
    )
    out_shapes = [
        jax.ShapeDtypeStruct((nsteps, 1, _TB), jnp.int32),
        jax.ShapeDtypeStruct((1, 1), jnp.float32),
    ]
    body = functools.partial(_encode_body, nsteps, 1.0 / (T * cb_dim))
    idx, loss = pl.pallas_call(body, grid_spec=grid_spec, out_shape=out_shapes)(
        x2, W1.T, b1.reshape(1, -1), W2.T, b2.reshape(1, -1),
        codebook.T, cbsq)
    return idx.reshape(T), loss


def _sc_gather(codebook, idx_flat):
    T = idx_flat.shape[0]
    cb_size, cb_dim = codebook.shape
    info = plsc.get_sparse_core_info()
    nw = info.num_cores * info.num_subcores
    b_per_w = T // nw
    mesh = plsc.VectorSubcoreMesh(core_axis_name="c", subcore_axis_name="s")

    lanes = 128
    cb_pad = jnp.zeros((cb_size, lanes), jnp.float32).at[:, :cb_dim].set(codebook)

    @functools.partial(
        pl.kernel, mesh=mesh,
        out_type=jax.ShapeDtypeStruct((T, lanes), jnp.float32),
        scratch_types=[
            pltpu.VMEM((b_per_w,), jnp.int32),
            pltpu.VMEM((b_per_w, lanes), jnp.float32),
            pltpu.SemaphoreType.DMA,
        ],
    )
    def gather(table_hbm, idx_hbm, out_hbm, idx_v, rows_v, sem):
        wid = lax.axis_index("s") * info.num_cores + lax.axis_index("c")
        base = wid * b_per_w
        pltpu.sync_copy(idx_hbm.at[pl.ds(base, b_per_w)], idx_v)
        pltpu.async_copy(table_hbm.at[idx_v], rows_v, sem).wait()
        pltpu.sync_copy(rows_v, out_hbm.at[pl.ds(base, b_per_w)])

    return gather(cb_pad, idx_flat)


def _decode_body(zq_ref, w3t_ref, b3_ref, w4t_ref, b4_ref, xrec_ref):
    h2 = _gelu_exact(jnp.dot(zq_ref[...], w3t_ref[...],
                             preferred_element_type=jnp.float32) + b3_ref[...])
    xrec_ref[...] = jnp.dot(h2, w4t_ref[...],
                            preferred_element_type=jnp.float32) + b4_ref[...]


def _decode(z_q, cb_dim, W3, b3, W4, b4):
    T, lanes = z_q.shape
    D = W4.shape[0]
    nsteps = T // _TBD
    w3t_pad = jnp.zeros((lanes, W3.shape[0]), jnp.float32).at[:cb_dim, :].set(W3.T)
    full = lambda i: (0, 0)
    grid_spec = pl.GridSpec(
        grid=(nsteps,),
        in_specs=[
            pl.BlockSpec((_TBD, lanes), lambda i: (i, 0)),
            pl.BlockSpec((lanes, W3.shape[0]), full),
            pl.BlockSpec((1, W3.shape[0]), full),
            pl.BlockSpec((W3.shape[0], D), full),
            pl.BlockSpec((1, D), full),
        ],
        out_specs=pl.BlockSpec((_TBD, D), lambda i: (i, 0)),
    )
    return pl.pallas_call(
        _decode_body,
        grid_spec=grid_spec,
        out_shape=jax.ShapeDtypeStruct((T, D), jnp.float32),
    )(z_q, w3t_pad, b3.reshape(1, -1), W4.T, b4.reshape(1, -1))


def kernel(x, W1, b1, W2, b2, codebook, W3, b3, W4, b4):
    B, N, D = x.shape
    T = B * N
    x2 = x.reshape(T, D)
    half = T // 2
    idx0, loss0 = _encode(x2[:half], W1, b1, W2, b2, codebook)
    zq0 = _sc_gather(codebook, idx0)
    idx1, loss1 = _encode(x2[half:], W1, b1, W2, b2, codebook)
    zq1 = _sc_gather(codebook, idx1)
    xr0 = _decode(zq0, codebook.shape[1], W3, b3, W4, b4)
    xr1 = _decode(zq1, codebook.shape[1], W3, b3, W4, b4)
    xrec = jnp.concatenate([xr0, xr1], axis=0)
    idx_flat = jnp.concatenate([idx0, idx1], axis=0)
    loss = (0.5 * (loss0 + loss1)).reshape(())
    return (xrec.reshape(B, N, D), idx_flat.reshape(B, N), loss)


# speedup vs baseline: 1.5823x; 1.5823x over previous
import functools

import jax
import jax.numpy as jnp
from jax import lax
from jax.experimental import pallas as pl
from jax.experimental.pallas import tpu as pltpu
from jax.experimental.pallas import tpu_sc as plsc

_TB = 1536
_SUB = 3
_TBD = 2304

_INV_SQRT2 = 0.7071067811865476


def _gelu_exact(v):
    return 0.5 * v * (1.0 + jax.lax.erf(v * _INV_SQRT2))


def _encode_body(nsteps, inv_count,
                 x_ref, w1t_ref, b1_ref, w2t_ref, b2_ref, cbt_ref, cbsq_ref,
                 idx_ref, loss_ref):
    i = pl.program_id(0)
    cs = _TB // _SUB
    parts = []
    for c in range(_SUB):
        rows = pl.ds(c * cs, cs)
        h = _gelu_exact(jnp.dot(x_ref[rows, :], w1t_ref[...],
                                preferred_element_type=jnp.float32) + b1_ref[...])
        z = jnp.dot(h, w2t_ref[...],
                    preferred_element_type=jnp.float32) + b2_ref[...]

        zsq = jnp.sum(z * z, axis=1, keepdims=True)
        zc2 = jnp.dot(2.0 * z, cbt_ref[...], preferred_element_type=jnp.float32)
        dist = (zsq + cbsq_ref[...]) - zc2

        m = jnp.min(dist, axis=1, keepdims=True)
        col = jax.lax.broadcasted_iota(jnp.int32, dist.shape, 1)
        idx = jnp.min(jnp.where(dist <= m, col, dist.shape[1]), axis=1)
        idx_ref[0, 0, rows] = idx
        parts.append(jnp.sum(m))

    part = sum(parts).reshape(1, 1)

    @pl.when(i == 0)
    def _():
        loss_ref[...] = jnp.zeros_like(loss_ref)

    loss_ref[...] += part

    @pl.when(i == nsteps - 1)
    def _():
        loss_ref[...] = loss_ref[...] * (1.25 * inv_count)


def _encode(x2, W1, b1, W2, b2, codebook):
    T, D = x2.shape
    cb_size, cb_dim = codebook.shape
    nsteps = T // _TB
    cbsq = jnp.sum(codebook * codebook, axis=1).reshape(1, cb_size)
    full = lambda i: (0, 0)
    grid_spec = pl.GridSpec(
        grid=(nsteps,),
        in_specs=[
            pl.BlockSpec((_TB, D), lambda i: (i, 0)),
            pl.BlockSpec((D, W1.shape[0]), full),
            pl.BlockSpec((1, W1.shape[0]), full),
            pl.BlockSpec((W1.shape[0], cb_dim), full),
            pl.BlockSpec((1, cb_dim), full),
            pl.BlockSpec((cb_dim, cb_size), full),
            pl.BlockSpec((1, cb_size), full),
        ],
        out_specs=[
            pl.BlockSpec((1, 1, _TB), lambda i: (i, 0, 0)),
            pl.BlockSpec((1, 1), full),
        ],
    )
    out_shapes = [
        jax.ShapeDtypeStruct((nsteps, 1, _TB), jnp.int32),
        jax.ShapeDtypeStruct((1, 1), jnp.float32),
    ]
    body = functools.partial(_encode_body, nsteps, 1.0 / (T * cb_dim))
    idx, loss = pl.pallas_call(body, grid_spec=grid_spec, out_shape=out_shapes)(
        x2, W1.T, b1.reshape(1, -1), W2.T, b2.reshape(1, -1),
        codebook.T, cbsq)
    return idx.reshape(T), loss


def _sc_gather(codebook, idx_flat):
    T = idx_flat.shape[0]
    cb_size, cb_dim = codebook.shape
    info = plsc.get_sparse_core_info()
    nw = info.num_cores * info.num_subcores
    b_per_w = T // nw
    mesh = plsc.VectorSubcoreMesh(core_axis_name="c", subcore_axis_name="s")

    lanes = 128
    cb_pad = jnp.zeros((cb_size, lanes), jnp.float32).at[:, :cb_dim].set(codebook)

    nchunk = 4
    ck = b_per_w // nchunk

    @functools.partial(
        pl.kernel, mesh=mesh,
        out_type=jax.ShapeDtypeStruct((T, lanes), jnp.float32),
        scratch_types=[
            pltpu.VMEM((b_per_w,), jnp.int32),
            pltpu.VMEM((ck, lanes), jnp.float32),
            pltpu.VMEM((ck, lanes), jnp.float32),
            pltpu.SemaphoreType.DMA,
            pltpu.SemaphoreType.DMA,
            pltpu.SemaphoreType.DMA,
            pltpu.SemaphoreType.DMA,
        ],
    )
    def gather(table_hbm, idx_hbm, out_hbm, idx_v, buf0, buf1,
               gs0, gs1, ss0, ss1):
        wid = lax.axis_index("s") * info.num_cores + lax.axis_index("c")
        base = wid * b_per_w
        bufs, gsems, ssems = (buf0, buf1), (gs0, gs1), (ss0, ss1)
        pltpu.sync_copy(idx_hbm.at[pl.ds(base, b_per_w)], idx_v)
        ghs, shs = [None] * nchunk, [None] * nchunk
        ghs[0] = pltpu.async_copy(
            table_hbm.at[idx_v.at[pl.ds(0, ck)]], bufs[0], gsems[0])
        for j in range(nchunk):
            b = j % 2
            if j + 1 < nchunk:
                if j >= 1:
                    shs[j - 1].wait()
                ghs[j + 1] = pltpu.async_copy(
                    table_hbm.at[idx_v.at[pl.ds((j + 1) * ck, ck)]],
                    bufs[1 - b], gsems[1 - b])
            ghs[j].wait()
            shs[j] = pltpu.async_copy(
                bufs[b], out_hbm.at[pl.ds(base + j * ck, ck)], ssems[b])
        shs[nchunk - 2].wait()
        shs[nchunk - 1].wait()

    return gather(cb_pad, idx_flat)


def _decode_body(zq_ref, w3t_ref, b3_ref, w4t_ref, b4_ref, xrec_ref):
    h2 = _gelu_exact(jnp.dot(zq_ref[...], w3t_ref[...],
                             preferred_element_type=jnp.float32) + b3_ref[...])
    xrec_ref[...] = jnp.dot(h2, w4t_ref[...],
                            preferred_element_type=jnp.float32) + b4_ref[...]


def _decode(z_q, cb_dim, W3, b3, W4, b4):
    T, lanes = z_q.shape
    D = W4.shape[0]
    nsteps = T // _TBD
    w3t_pad = jnp.zeros((lanes, W3.shape[0]), jnp.float32).at[:cb_dim, :].set(W3.T)
    full = lambda i: (0, 0)
    grid_spec = pl.GridSpec(
        grid=(nsteps,),
        in_specs=[
            pl.BlockSpec((_TBD, lanes), lambda i: (i, 0)),
            pl.BlockSpec((lanes, W3.shape[0]), full),
            pl.BlockSpec((1, W3.shape[0]), full),
            pl.BlockSpec((W3.shape[0], D), full),
            pl.BlockSpec((1, D), full),
        ],
        out_specs=pl.BlockSpec((_TBD, D), lambda i: (i, 0)),
    )
    return pl.pallas_call(
        _decode_body,
        grid_spec=grid_spec,
        out_shape=jax.ShapeDtypeStruct((T, D), jnp.float32),
    )(z_q, w3t_pad, b3.reshape(1, -1), W4.T, b4.reshape(1, -1))


def kernel(x, W1, b1, W2, b2, codebook, W3, b3, W4, b4):
    B, N, D = x.shape
    T = B * N
    x2 = x.reshape(T, D)
    idx_flat, loss = _encode(x2, W1, b1, W2, b2, codebook)
    z_q = _sc_gather(codebook, idx_flat)
    xrec = _decode(z_q, codebook.shape[1], W3, b3, W4, b4)
    return (xrec.reshape(B, N, D), idx_flat.reshape(B, N), loss.reshape(()))


# speedup vs baseline: 1.6483x; 1.0417x over previous
import functools

import jax
import jax.numpy as jnp
from jax import lax
from jax.experimental import pallas as pl
from jax.experimental.pallas import tpu as pltpu
from jax.experimental.pallas import tpu_sc as plsc

_TB = 1536
_SUB = 3
_TBD = 2304

_INV_SQRT2 = 0.7071067811865476


def _gelu_exact(v):
    return 0.5 * v * (1.0 + jax.lax.erf(v * _INV_SQRT2))


def _encode_body(nsteps, inv_count,
                 x_ref, w1t_ref, b1_ref, w2t_ref, b2_ref, cbt_ref, cbsq_ref,
                 cb_ref, w3t_ref, b3_ref,
                 idx_ref, loss_ref, h2cb_ref):
    i = pl.program_id(0)
    cs = _TB // _SUB

    @pl.when(i == 0)
    def _():
        h2cb_ref[...] = _gelu_exact(
            jnp.dot(cb_ref[...], w3t_ref[...],
                    preferred_element_type=jnp.float32) + b3_ref[...])

    parts = []
    for c in range(_SUB):
        rows = pl.ds(c * cs, cs)
        h = _gelu_exact(jnp.dot(x_ref[rows, :], w1t_ref[...],
                                preferred_element_type=jnp.float32) + b1_ref[...])
        z = jnp.dot(h, w2t_ref[...],
                    preferred_element_type=jnp.float32) + b2_ref[...]

        zsq = jnp.sum(z * z, axis=1, keepdims=True)
        zc2 = jnp.dot(2.0 * z, cbt_ref[...], preferred_element_type=jnp.float32)
        dist = (zsq + cbsq_ref[...]) - zc2

        m = jnp.min(dist, axis=1, keepdims=True)
        col = jax.lax.broadcasted_iota(jnp.int32, dist.shape, 1)
        idx = jnp.min(jnp.where(dist <= m, col, dist.shape[1]), axis=1)
        idx_ref[0, 0, rows] = idx
        parts.append(jnp.sum(m))

    part = sum(parts).reshape(1, 1)

    @pl.when(i == 0)
    def _():
        loss_ref[...] = jnp.zeros_like(loss_ref)

    loss_ref[...] += part

    @pl.when(i == nsteps - 1)
    def _():
        loss_ref[...] = loss_ref[...] * (1.25 * inv_count)


def _encode(x2, W1, b1, W2, b2, codebook, W3, b3):
    T, D = x2.shape
    cb_size, cb_dim = codebook.shape
    dh = W3.shape[0]
    nsteps = T // _TB
    cbsq = jnp.sum(codebook * codebook, axis=1).reshape(1, cb_size)
    full = lambda i: (0, 0)
    grid_spec = pl.GridSpec(
        grid=(nsteps,),
        in_specs=[
            pl.BlockSpec((_TB, D), lambda i: (i, 0)),
            pl.BlockSpec((D, W1.shape[0]), full),
            pl.BlockSpec((1, W1.shape[0]), full),
            pl.BlockSpec((W1.shape[0], cb_dim), full),
            pl.BlockSpec((1, cb_dim), full),
            pl.BlockSpec((cb_dim, cb_size), full),
            pl.BlockSpec((1, cb_size), full),
            pl.BlockSpec((cb_size, cb_dim), full),
            pl.BlockSpec((cb_dim, dh), full),
            pl.BlockSpec((1, dh), full),
        ],
        out_specs=[
            pl.BlockSpec((1, 1, _TB), lambda i: (i, 0, 0)),
            pl.BlockSpec((1, 1), full),
            pl.BlockSpec((cb_size, dh), full),
        ],
    )
    out_shapes = [
        jax.ShapeDtypeStruct((nsteps, 1, _TB), jnp.int32),
        jax.ShapeDtypeStruct((1, 1), jnp.float32),
        jax.ShapeDtypeStruct((cb_size, dh), jnp.float32),
    ]
    body = functools.partial(_encode_body, nsteps, 1.0 / (T * cb_dim))
    idx, loss, h2cb = pl.pallas_call(
        body, grid_spec=grid_spec, out_shape=out_shapes)(
        x2, W1.T, b1.reshape(1, -1), W2.T, b2.reshape(1, -1),
        codebook.T, cbsq, codebook, W3.T, b3.reshape(1, -1))
    return idx.reshape(T), loss, h2cb


def _sc_gather(table, idx_flat):
    T = idx_flat.shape[0]
    n_rows, lanes = table.shape
    info = plsc.get_sparse_core_info()
    nw = info.num_cores * info.num_subcores
    b_per_w = T // nw
    mesh = plsc.VectorSubcoreMesh(core_axis_name="c", subcore_axis_name="s")

    @functools.partial(
        pl.kernel, mesh=mesh,
        out_type=jax.ShapeDtypeStruct((T, lanes), jnp.float32),
        scratch_types=[
            pltpu.VMEM((b_per_w,), jnp.int32),
            pltpu.VMEM((b_per_w, lanes), jnp.float32),
            pltpu.SemaphoreType.DMA,
        ],
    )
    def gather(table_hbm, idx_hbm, out_hbm, idx_v, rows_v, sem):
        wid = lax.axis_index("s") * info.num_cores + lax.axis_index("c")
        base = wid * b_per_w
        pltpu.sync_copy(idx_hbm.at[pl.ds(base, b_per_w)], idx_v)
        pltpu.async_copy(table_hbm.at[idx_v], rows_v, sem).wait()
        pltpu.sync_copy(rows_v, out_hbm.at[pl.ds(base, b_per_w)])

    return gather(table, idx_flat)


def _decode_body(h2q_ref, w4t_ref, b4_ref, xrec_ref):
    xrec_ref[...] = jnp.dot(h2q_ref[...], w4t_ref[...],
                            preferred_element_type=jnp.float32) + b4_ref[...]


def _decode(h2q, W4, b4):
    T, dh = h2q.shape
    D = W4.shape[0]
    nsteps = T // _TBD
    full = lambda i: (0, 0)
    grid_spec = pl.GridSpec(
        grid=(nsteps,),
        in_specs=[
            pl.BlockSpec((_TBD, dh), lambda i: (i, 0)),
            pl.BlockSpec((dh, D), full),
            pl.BlockSpec((1, D), full),
        ],
        out_specs=pl.BlockSpec((_TBD, D), lambda i: (i, 0)),
    )
    return pl.pallas_call(
        _decode_body,
        grid_spec=grid_spec,
        out_shape=jax.ShapeDtypeStruct((T, D), jnp.float32),
    )(h2q, W4.T, b4.reshape(1, -1))


def kernel(x, W1, b1, W2, b2, codebook, W3, b3, W4, b4):
    B, N, D = x.shape
    T = B * N
    x2 = x.reshape(T, D)
    idx_flat, loss, h2cb = _encode(x2, W1, b1, W2, b2, codebook, W3, b3)
    h2q = _sc_gather(h2cb, idx_flat)
    xrec = _decode(h2q, W4, b4)
    return (xrec.reshape(B, N, D), idx_flat.reshape(B, N), loss.reshape(()))


# speedup vs baseline: 1.6801x; 1.0193x over previous
import functools

import jax
import jax.numpy as jnp
from jax import lax
from jax.experimental import pallas as pl
from jax.experimental.pallas import tpu as pltpu
from jax.experimental.pallas import tpu_sc as plsc

_TB = 2048
_SUB = 4
_TBD = 2304

_INV_SQRT2 = 0.7071067811865476


def _gelu_exact(v):
    return 0.5 * v * (1.0 + jax.lax.erf(v * _INV_SQRT2))


def _encode_body(nsteps, inv_count,
                 x_ref, w1t_ref, b1_ref, w2t_ref, b2_ref, cbt_ref, cbsq_ref,
                 cb_ref, w3t_ref, b3_ref,
                 idx_ref, loss_ref, h2cb_ref):
    i = pl.program_id(0)
    cs = _TB // _SUB

    @pl.when(i == 0)
    def _():
        h2cb_ref[...] = _gelu_exact(
            jnp.dot(cb_ref[...], w3t_ref[...],
                    preferred_element_type=jnp.float32) + b3_ref[...])

    parts = []
    for c in range(_SUB):
        rows = pl.ds(c * cs, cs)
        h = _gelu_exact(jnp.dot(x_ref[rows, :], w1t_ref[...],
                                preferred_element_type=jnp.float32) + b1_ref[...])
        z = jnp.dot(h, w2t_ref[...],
                    preferred_element_type=jnp.float32) + b2_ref[...]

        zsq = jnp.sum(z * z, axis=1, keepdims=True)
        zc2 = jnp.dot(2.0 * z, cbt_ref[...], preferred_element_type=jnp.float32)
        dist = (zsq + cbsq_ref[...]) - zc2

        m = jnp.min(dist, axis=1, keepdims=True)
        col = jax.lax.broadcasted_iota(jnp.int32, dist.shape, 1)
        idx = jnp.min(jnp.where(dist <= m, col, dist.shape[1]), axis=1)
        idx_ref[0, 0, rows] = idx
        parts.append(jnp.sum(m))

    part = sum(parts).reshape(1, 1)

    @pl.when(i == 0)
    def _():
        loss_ref[...] = jnp.zeros_like(loss_ref)

    loss_ref[...] += part

    @pl.when(i == nsteps - 1)
    def _():
        loss_ref[...] = loss_ref[...] * (1.25 * inv_count)


def _encode(x2, W1, b1, W2, b2, codebook, W3, b3):
    T, D = x2.shape
    cb_size, cb_dim = codebook.shape
    dh = W3.shape[0]
    nsteps = T // _TB
    cbsq = jnp.sum(codebook * codebook, axis=1).reshape(1, cb_size)
    full = lambda i: (0, 0)
    grid_spec = pl.GridSpec(
        grid=(nsteps,),
        in_specs=[
            pl.BlockSpec((_TB, D), lambda i: (i, 0)),
            pl.BlockSpec((D, W1.shape[0]), full),
            pl.BlockSpec((1, W1.shape[0]), full),
            pl.BlockSpec((W1.shape[0], cb_dim), full),
            pl.BlockSpec((1, cb_dim), full),
            pl.BlockSpec((cb_dim, cb_size), full),
            pl.BlockSpec((1, cb_size), full),
            pl.BlockSpec((cb_size, cb_dim), full),
            pl.BlockSpec((cb_dim, dh), full),
            pl.BlockSpec((1, dh), full),
        ],
        out_specs=[
            pl.BlockSpec((1, 1, _TB), lambda i: (i, 0, 0)),
            pl.BlockSpec((1, 1), full),
            pl.BlockSpec((cb_size, dh), full),
        ],
    )
    out_shapes = [
        jax.ShapeDtypeStruct((nsteps, 1, _TB), jnp.int32),
        jax.ShapeDtypeStruct((1, 1), jnp.float32),
        jax.ShapeDtypeStruct((cb_size, dh), jnp.float32),
    ]
    body = functools.partial(_encode_body, nsteps, 1.0 / (T * cb_dim))
    idx, loss, h2cb = pl.pallas_call(
        body, grid_spec=grid_spec, out_shape=out_shapes)(
        x2, W1.T, b1.reshape(1, -1), W2.T, b2.reshape(1, -1),
        codebook.T, cbsq, codebook, W3.T, b3.reshape(1, -1))
    return idx.reshape(T), loss, h2cb


def _sc_gather(table, idx_flat):
    T = idx_flat.shape[0]
    n_rows, lanes = table.shape
    info = plsc.get_sparse_core_info()
    nw = info.num_cores * info.num_subcores
    b_per_w = T // nw
    mesh = plsc.VectorSubcoreMesh(core_axis_name="c", subcore_axis_name="s")

    @functools.partial(
        pl.kernel, mesh=mesh,
        out_type=jax.ShapeDtypeStruct((T, lanes), jnp.float32),
        scratch_types=[
            pltpu.VMEM((b_per_w,), jnp.int32),
            pltpu.VMEM((b_per_w, lanes), jnp.float32),
            pltpu.SemaphoreType.DMA,
        ],
    )
    def gather(table_hbm, idx_hbm, out_hbm, idx_v, rows_v, sem):
        wid = lax.axis_index("s") * info.num_cores + lax.axis_index("c")
        base = wid * b_per_w
        pltpu.sync_copy(idx_hbm.at[pl.ds(base, b_per_w)], idx_v)
        pltpu.async_copy(table_hbm.at[idx_v], rows_v, sem).wait()
        pltpu.sync_copy(rows_v, out_hbm.at[pl.ds(base, b_per_w)])

    return gather(table, idx_flat)


def _decode_body(h2q_ref, w4t_ref, b4_ref, xrec_ref):
    xrec_ref[...] = jnp.dot(h2q_ref[...], w4t_ref[...],
                            preferred_element_type=jnp.float32) + b4_ref[...]


def _decode(h2q, W4, b4):
    T, dh = h2q.shape
    D = W4.shape[0]
    nsteps = T // _TBD
    full = lambda i: (0, 0)
    grid_spec = pl.GridSpec(
        grid=(nsteps,),
        in_specs=[
            pl.BlockSpec((_TBD, dh), lambda i: (i, 0)),
            pl.BlockSpec((dh, D), full),
            pl.BlockSpec((1, D), full),
        ],
        out_specs=pl.BlockSpec((_TBD, D), lambda i: (i, 0)),
    )
    return pl.pallas_call(
        _decode_body,
        grid_spec=grid_spec,
        out_shape=jax.ShapeDtypeStruct((T, D), jnp.float32),
    )(h2q, W4.T, b4.reshape(1, -1))


def kernel(x, W1, b1, W2, b2, codebook, W3, b3, W4, b4):
    B, N, D = x.shape
    T = B * N
    x2 = x.reshape(T, D)
    idx_flat, loss, h2cb = _encode(x2, W1, b1, W2, b2, codebook, W3, b3)
    h2q = _sc_gather(h2cb, idx_flat)
    xrec = _decode(h2q, W4, b4)
    return (xrec.reshape(B, N, D), idx_flat.reshape(B, N), loss.reshape(()))


# speedup vs baseline: 1.6960x; 1.0095x over previous
import functools

import jax
import jax.numpy as jnp
from jax import lax
from jax.experimental import pallas as pl
from jax.experimental.pallas import tpu as pltpu
from jax.experimental.pallas import tpu_sc as plsc

_TB = 3072
_SUB = 6
_TBD = 2304

_INV_SQRT2 = 0.7071067811865476


def _gelu_exact(v):
    return 0.5 * v * (1.0 + jax.lax.erf(v * _INV_SQRT2))


def _encode_body(nsteps, inv_count,
                 x_ref, w1t_ref, b1_ref, w2t_ref, b2_ref, cbt_ref, cbsq_ref,
                 cb_ref, w3t_ref, b3_ref,
                 idx_ref, loss_ref, h2cb_ref):
    i = pl.program_id(0)
    cs = _TB // _SUB

    @pl.when(i == 0)
    def _():
        h2cb_ref[...] = _gelu_exact(
            jnp.dot(cb_ref[...], w3t_ref[...],
                    preferred_element_type=jnp.float32) + b3_ref[...])

    parts = []
    for c in range(_SUB):
        rows = pl.ds(c * cs, cs)
        h = _gelu_exact(jnp.dot(x_ref[rows, :], w1t_ref[...],
                                preferred_element_type=jnp.float32) + b1_ref[...])
        z = jnp.dot(h, w2t_ref[...],
                    preferred_element_type=jnp.float32) + b2_ref[...]

        zsq = jnp.sum(z * z, axis=1, keepdims=True)
        zc2 = jnp.dot(2.0 * z, cbt_ref[...], preferred_element_type=jnp.float32)
        dist = (zsq + cbsq_ref[...]) - zc2

        m = jnp.min(dist, axis=1, keepdims=True)
        col = jax.lax.broadcasted_iota(jnp.int32, dist.shape, 1)
        idx = jnp.min(jnp.where(dist <= m, col, dist.shape[1]), axis=1)
        idx_ref[0, 0, rows] = idx
        parts.append(jnp.sum(m))

    part = sum(parts).reshape(1, 1)

    @pl.when(i == 0)
    def _():
        loss_ref[...] = jnp.zeros_like(loss_ref)

    loss_ref[...] += part

    @pl.when(i == nsteps - 1)
    def _():
        loss_ref[...] = loss_ref[...] * (1.25 * inv_count)


def _encode(x2, W1, b1, W2, b2, codebook, W3, b3):
    T, D = x2.shape
    cb_size, cb_dim = codebook.shape
    dh = W3.shape[0]
    nsteps = T // _TB
    cbsq = jnp.sum(codebook * codebook, axis=1).reshape(1, cb_size)
    full = lambda i: (0, 0)
    grid_spec = pl.GridSpec(
        grid=(nsteps,),
        in_specs=[
            pl.BlockSpec((_TB, D), lambda i: (i, 0)),
            pl.BlockSpec((D, W1.shape[0]), full),
            pl.BlockSpec((1, W1.shape[0]), full),
            pl.BlockSpec((W1.shape[0], cb_dim), full),
            pl.BlockSpec((1, cb_dim), full),
            pl.BlockSpec((cb_dim, cb_size), full),
            pl.BlockSpec((1, cb_size), full),
            pl.BlockSpec((cb_size, cb_dim), full),
            pl.BlockSpec((cb_dim, dh), full),
            pl.BlockSpec((1, dh), full),
        ],
        out_specs=[
            pl.BlockSpec((1, 1, _TB), lambda i: (i, 0, 0)),
            pl.BlockSpec((1, 1), full),
            pl.BlockSpec((cb_size, dh), full),
        ],
    )
    out_shapes = [
        jax.ShapeDtypeStruct((nsteps, 1, _TB), jnp.int32),
        jax.ShapeDtypeStruct((1, 1), jnp.float32),
        jax.ShapeDtypeStruct((cb_size, dh), jnp.float32),
    ]
    body = functools.partial(_encode_body, nsteps, 1.0 / (T * cb_dim))
    idx, loss, h2cb = pl.pallas_call(
        body, grid_spec=grid_spec, out_shape=out_shapes)(
        x2, W1.T, b1.reshape(1, -1), W2.T, b2.reshape(1, -1),
        codebook.T, cbsq, codebook, W3.T, b3.reshape(1, -1))
    return idx.reshape(T), loss, h2cb


def _sc_gather(table, idx_flat):
    T = idx_flat.shape[0]
    n_rows, lanes = table.shape
    info = plsc.get_sparse_core_info()
    nw = info.num_cores * info.num_subcores
    b_per_w = T // nw
    mesh = plsc.VectorSubcoreMesh(core_axis_name="c", subcore_axis_name="s")

    @functools.partial(
        pl.kernel, mesh=mesh,
        out_type=jax.ShapeDtypeStruct((T, lanes), jnp.float32),
        scratch_types=[
            pltpu.VMEM((b_per_w,), jnp.int32),
            pltpu.VMEM((b_per_w, lanes), jnp.float32),
            pltpu.SemaphoreType.DMA,
        ],
    )
    def gather(table_hbm, idx_hbm, out_hbm, idx_v, rows_v, sem):
        wid = lax.axis_index("s") * info.num_cores + lax.axis_index("c")
        base = wid * b_per_w
        pltpu.sync_copy(idx_hbm.at[pl.ds(base, b_per_w)], idx_v)
        pltpu.async_copy(table_hbm.at[idx_v], rows_v, sem).wait()
        pltpu.sync_copy(rows_v, out_hbm.at[pl.ds(base, b_per_w)])

    return gather(table, idx_flat)


def _decode_body(h2q_ref, w4t_ref, b4_ref, xrec_ref):
    xrec_ref[...] = jnp.dot(h2q_ref[...], w4t_ref[...],
                            preferred_element_type=jnp.float32) + b4_ref[...]


def _decode(h2q, W4, b4):
    T, dh = h2q.shape
    D = W4.shape[0]
    nsteps = T // _TBD
    full = lambda i: (0, 0)
    grid_spec = pl.GridSpec(
        grid=(nsteps,),
        in_specs=[
            pl.BlockSpec((_TBD, dh), lambda i: (i, 0)),
            pl.BlockSpec((dh, D), full),
            pl.BlockSpec((1, D), full),
        ],
        out_specs=pl.BlockSpec((_TBD, D), lambda i: (i, 0)),
    )
    return pl.pallas_call(
        _decode_body,
        grid_spec=grid_spec,
        out_shape=jax.ShapeDtypeStruct((T, D), jnp.float32),
    )(h2q, W4.T, b4.reshape(1, -1))


def kernel(x, W1, b1, W2, b2, codebook, W3, b3, W4, b4):
    B, N, D = x.shape
    T = B * N
    x2 = x.reshape(T, D)
    idx_flat, loss, h2cb = _encode(x2, W1, b1, W2, b2, codebook, W3, b3)
    h2q = _sc_gather(h2cb, idx_flat)
    xrec = _decode(h2q, W4, b4)
    return (xrec.reshape(B, N, D), idx_flat.reshape(B, N), loss.reshape(()))
